# baseline (speedup 1.0000x reference)
import functools

import jax
import jax.numpy as jnp
from jax import lax
from jax.experimental import pallas as pl
from jax.experimental.pallas import tpu as pltpu
from jax.experimental.pallas import tpu_sc as plsc

BATCH = 16384
EMB = 16
NUM_CORES = 2
NUM_SUBCORES = 16
NUM_WORKERS = NUM_CORES * NUM_SUBCORES
ROWS_PER_WORKER = BATCH // NUM_WORKERS


def kernel(user_idx, item_idx, user_emb, item_emb):
    mesh = plsc.VectorSubcoreMesh(core_axis_name="c", subcore_axis_name="s")

    @functools.partial(
        pl.kernel,
        out_type=jax.ShapeDtypeStruct((BATCH, EMB), jnp.float32),
        mesh=mesh,
        compiler_params=pltpu.CompilerParams(use_tc_tiling_on_sc=False),
        scratch_types=[
            pltpu.VMEM((ROWS_PER_WORKER,), jnp.int32),
            pltpu.VMEM((ROWS_PER_WORKER,), jnp.int32),
            pltpu.VMEM((ROWS_PER_WORKER, EMB), jnp.float32),
            pltpu.VMEM((ROWS_PER_WORKER, EMB), jnp.float32),
            pltpu.SemaphoreType.DMA,
            pltpu.SemaphoreType.DMA,
        ],
    )
    def gmf(uidx_hbm, iidx_hbm, uemb_hbm, iemb_hbm, out_hbm,
            uidx_v, iidx_v, u_v, i_v, sem_u, sem_i):
        wid = lax.axis_index("s") * NUM_CORES + lax.axis_index("c")
        base = wid * ROWS_PER_WORKER
        pltpu.sync_copy(uidx_hbm.at[pl.ds(base, ROWS_PER_WORKER)], uidx_v)
        pltpu.sync_copy(iidx_hbm.at[pl.ds(base, ROWS_PER_WORKER)], iidx_v)
        cu = pltpu.async_copy(uemb_hbm.at[uidx_v], u_v, sem_u)
        ci = pltpu.async_copy(iemb_hbm.at[iidx_v], i_v, sem_i)
        cu.wait()
        ci.wait()

        @pl.loop(0, ROWS_PER_WORKER)
        def _(r):
            u_v[r, :] = u_v[r, :] * i_v[r, :]

        pltpu.sync_copy(u_v, out_hbm.at[pl.ds(base, ROWS_PER_WORKER)])

    return gmf(user_idx, item_idx, user_emb, item_emb)
